# baseline (speedup 1.0000x reference)
import functools

import jax
import jax.numpy as jnp
from jax import lax
from jax.experimental import pallas as pl
from jax.experimental.pallas import tpu as pltpu
from jax.experimental.pallas import tpu_sc as plsc

K = 8
D = 128
BR = 256
NC, NS, L = 2, 16, 16
CR = 8


def _topk_body(row_offset, x_ref, xt_ref, dis_ref, idx_ref, wsp_ref):
    n = xt_ref.shape[1]
    pid = pl.program_id(0)
    x = x_ref[...]
    xt = xt_ref[...]
    sq_r = jnp.sum(x * x, axis=1, keepdims=True)
    sq_c = jnp.sum(xt * xt, axis=0, keepdims=True)
    g = jnp.dot(x, xt, preferred_element_type=jnp.float32)
    dist = jnp.maximum(sq_r + sq_c - 2.0 * g, 0.0)
    col = lax.broadcasted_iota(jnp.int32, (BR, n), 1)
    row = (lax.broadcasted_iota(jnp.int32, (BR, n), 0)
           + pid * BR + row_offset)
    dist = jnp.where(col == row, jnp.float32(1e10), dist)
    m0 = jnp.min(dist, axis=1, keepdims=True)
    rel = dist - (m0 - 1.0)
    kb = (lax.bitcast_convert_type(rel, jnp.int32) & jnp.int32(~8191)) | col
    keys = pltpu.bitcast(kb, jnp.float32)
    bigf = jnp.float32(3.0e38)
    dvals, ivals = [], []
    for _ in range(K):
        mk = jnp.min(keys, axis=1, keepdims=True)
        mkb = lax.bitcast_convert_type(mk, jnp.int32)
        dvals.append(m0 - 1.0 + pltpu.bitcast(
            mkb & jnp.int32(~8191), jnp.float32))
        ivals.append(mkb & jnp.int32(8191))
        keys = jnp.where(keys == mk, bigf, keys)
    dis = jnp.concatenate(dvals, axis=1)
    idx = jnp.concatenate(ivals, axis=1)
    dis_ref[...] = dis
    idx_ref[...] = idx
    e = jnp.exp(dis[:, 0:1] - dis)
    w = e / jnp.sum(e, axis=1, keepdims=True)
    kcol = lax.broadcasted_iota(jnp.int32, (BR, D), 1) // L
    wsp = jnp.zeros((BR, D), jnp.float32)
    for k in range(K):
        wsp = jnp.where(kcol == k, w[:, k:k + 1], wsp)
    wsp_ref[...] = wsp


def _topk(xq, xt, row_offset):
    nq = xq.shape[0]
    n = xt.shape[1]
    return pl.pallas_call(
        functools.partial(_topk_body, row_offset),
        grid=(nq // BR,),
        in_specs=[
            pl.BlockSpec((BR, D), lambda i: (i, 0)),
            pl.BlockSpec((D, n), lambda i: (0, 0)),
        ],
        out_specs=[
            pl.BlockSpec((BR, K), lambda i: (i, 0)),
            pl.BlockSpec((BR, K), lambda i: (i, 0)),
            pl.BlockSpec((BR, D), lambda i: (i, 0)),
        ],
        out_shape=[
            jax.ShapeDtypeStruct((nq, K), jnp.float32),
            jax.ShapeDtypeStruct((nq, K), jnp.int32),
            jax.ShapeDtypeStruct((nq, D), jnp.float32),
        ],
    )(xq, xt)


def _combine(gene, idx_flat, wsp):
    gdim = gene.shape[1]
    nq = wsp.shape[0]
    rows_per_w = nq // (NC * NS)
    nchunks = rows_per_w // CR
    mesh = plsc.VectorSubcoreMesh(core_axis_name="c", subcore_axis_name="s",
                                  num_cores=NC, num_subcores=NS)

    @functools.partial(
        pl.kernel,
        out_type=jax.ShapeDtypeStruct((nq, gdim), jnp.float32),
        mesh=mesh,
        scratch_types=[
            pltpu.VMEM((2, CR * K), jnp.int32),
            pltpu.VMEM((2, CR, D), jnp.float32),
            pltpu.VMEM((2, CR, gdim), jnp.float32),
            pltpu.VMEM((2, CR * K, gdim), jnp.float32),
            pltpu.VMEM((2, CR, gdim), jnp.float32),
            pltpu.SemaphoreType.DMA,
            pltpu.SemaphoreType.DMA,
            pltpu.SemaphoreType.DMA,
            pltpu.SemaphoreType.DMA,
            pltpu.SemaphoreType.DMA,
            pltpu.SemaphoreType.DMA,
        ],
    )
    def sc_kernel(gene_hbm, idx_hbm, wsp_hbm, out_hbm,
                  idx_v, w_v, own_v, rows_v, out_v,
                  sa0, sa1, sg0, sg1, so0, so1):
        wid = lax.axis_index("s") * NC + lax.axis_index("c")
        base = wid * rows_per_w
        sa, sg, so = (sa0, sa1), (sg0, sg1), (so0, so1)

        def in_copies(c, b):
            r0 = base + c * CR
            return (
                (idx_hbm.at[pl.ds(r0 * K, CR * K)], idx_v.at[b], sa[b]),
                (wsp_hbm.at[pl.ds(r0, CR)], w_v.at[b], sa[b]),
                (gene_hbm.at[pl.ds(r0, CR)], own_v.at[b], sa[b]),
            )

        def start_in(c, b):
            for src, dst, sem in in_copies(c, b):
                pltpu.async_copy(src, dst, sem)

        def wait_in(c, b):
            for src, dst, sem in in_copies(c, b):
                pltpu.make_async_copy(src, dst, sem).wait()

        def start_g(b):
            pltpu.async_copy(gene_hbm.at[idx_v.at[b]], rows_v.at[b], sg[b])

        def wait_g(b):
            pltpu.make_async_copy(gene_hbm.at[idx_v.at[b]], rows_v.at[b],
                                  sg[b]).wait()

        def out_copy(c, b):
            r0 = base + c * CR
            return (out_v.at[b], out_hbm.at[pl.ds(r0, CR)], so[b])

        def compute(b):
            for r in range(CR):
                wv = [w_v[b, r, pl.ds(k * L, L)] for k in range(K)]
                for dc in range(gdim // L):
                    dd = dc * L
                    acc = own_v[b, r, pl.ds(dd, L)]
                    for k in range(K):
                        acc = acc + wv[k] * rows_v[b, r * K + k,
                                                   pl.ds(dd, L)]
                    out_v[b, r, pl.ds(dd, L)] = acc

        start_in(0, 0)
        wait_in(0, 0)
        start_g(0)
        start_in(1, 1)

        def pair(p, _):
            for b in range(2):
                c = 2 * p + b
                nb = 1 - b

                @pl.when(c + 1 < nchunks)
                def _():
                    wait_in(c + 1, nb)
                    start_g(nb)

                wait_g(b)

                @pl.when(c >= 2)
                def _():
                    src, dst, sem = out_copy(c - 2, b)
                    pltpu.make_async_copy(src, dst, sem).wait()

                compute(b)
                src, dst, sem = out_copy(c, b)
                pltpu.async_copy(src, dst, sem)

                @pl.when(c + 2 < nchunks)
                def _():
                    start_in(c + 2, b)

            return 0

        lax.fori_loop(0, nchunks // 2, pair, 0)
        for b in range(2):
            c = nchunks - 2 + b
            src, dst, sem = out_copy(c, b)
            pltpu.make_async_copy(src, dst, sem).wait()

    return sc_kernel(gene, idx_flat, wsp)


NCHUNK = 1


def kernel(raw, img, adata, gene, num):
    n = raw.shape[0]
    x = jnp.concatenate([raw, img, adata], axis=1)
    x = jnp.pad(x, ((0, 0), (0, D - x.shape[1])))
    xt = x.T
    step = n // NCHUNK
    sups, diss = [], []
    token = jnp.int32(0)
    for c in range(NCHUNK):
        dis_c, idx_c, wsp_c = _topk(
            lax.slice_in_dim(x, c * step, (c + 1) * step), xt, c * step)
        sup_c = _combine(gene, idx_c.reshape(-1) + token, wsp_c)
        token = lax.convert_element_type(sup_c[0, 0], jnp.int32) * 0
        sups.append(sup_c)
        diss.append(dis_c)
    return jnp.concatenate(sups, axis=0), jnp.concatenate(diss, axis=0)


# speedup vs baseline: 1.3893x; 1.3893x over previous
import functools

import jax
import jax.numpy as jnp
from jax import lax
from jax.experimental import pallas as pl
from jax.experimental.pallas import tpu as pltpu
from jax.experimental.pallas import tpu_sc as plsc

K = 8
D = 128
BR = 256
NC, NS, L = 2, 16, 16
CR = 8


def _topk_body(row_offset, x_ref, xt_ref, dis_ref, idx_ref, wsp_ref):
    n = xt_ref.shape[1]
    pid = pl.program_id(0)
    x = x_ref[...]
    xt = xt_ref[...]
    sq_r = jnp.sum(x * x, axis=1, keepdims=True)
    sq_c = jnp.sum(xt * xt, axis=0, keepdims=True)
    g = jnp.dot(x, xt, preferred_element_type=jnp.float32)
    dist = jnp.maximum(sq_r + sq_c - 2.0 * g, 0.0)
    col = lax.broadcasted_iota(jnp.int32, (BR, n), 1)
    row = (lax.broadcasted_iota(jnp.int32, (BR, n), 0)
           + pid * BR + row_offset)
    dist = jnp.where(col == row, jnp.float32(1e10), dist)
    m0 = jnp.min(dist, axis=1, keepdims=True)
    rel = dist - (m0 - 1.0)
    kb = (lax.bitcast_convert_type(rel, jnp.int32) & jnp.int32(~8191)) | col
    keys = pltpu.bitcast(kb, jnp.float32)
    bigf = jnp.float32(3.0e38)
    dvals, ivals = [], []
    for _ in range(K):
        mk = jnp.min(keys, axis=1, keepdims=True)
        mkb = lax.bitcast_convert_type(mk, jnp.int32)
        dvals.append(m0 - 1.0 + pltpu.bitcast(
            mkb & jnp.int32(~8191), jnp.float32))
        ivals.append(mkb & jnp.int32(8191))
        keys = jnp.where(keys == mk, bigf, keys)
    dis = jnp.concatenate(dvals, axis=1)
    idx = jnp.concatenate(ivals, axis=1)
    dis_ref[...] = dis
    idx_ref[...] = idx
    e = jnp.exp(dis[:, 0:1] - dis)
    w = e / jnp.sum(e, axis=1, keepdims=True)
    kcol = lax.broadcasted_iota(jnp.int32, (BR, D), 1) // L
    wsp = jnp.zeros((BR, D), jnp.float32)
    for k in range(K):
        wsp = jnp.where(kcol == k, w[:, k:k + 1], wsp)
    wsp_ref[...] = wsp


def _topk(xq, xt, row_offset):
    nq = xq.shape[0]
    n = xt.shape[1]
    return pl.pallas_call(
        functools.partial(_topk_body, row_offset),
        grid=(nq // BR,),
        in_specs=[
            pl.BlockSpec((BR, D), lambda i: (i, 0)),
            pl.BlockSpec((D, n), lambda i: (0, 0)),
        ],
        out_specs=[
            pl.BlockSpec((BR, K), lambda i: (i, 0)),
            pl.BlockSpec((BR, K), lambda i: (i, 0)),
            pl.BlockSpec((BR, D), lambda i: (i, 0)),
        ],
        out_shape=[
            jax.ShapeDtypeStruct((nq, K), jnp.float32),
            jax.ShapeDtypeStruct((nq, K), jnp.int32),
            jax.ShapeDtypeStruct((nq, D), jnp.float32),
        ],
    )(xq, xt)


def _combine(gene, idx_flat, wsp):
    gdim = gene.shape[1]
    nq = wsp.shape[0]
    rows_per_w = nq // (NC * NS)
    nchunks = rows_per_w // CR
    mesh = plsc.VectorSubcoreMesh(core_axis_name="c", subcore_axis_name="s",
                                  num_cores=NC, num_subcores=NS)

    @functools.partial(
        pl.kernel,
        out_type=jax.ShapeDtypeStruct((nq, gdim), jnp.float32),
        mesh=mesh,
        scratch_types=[
            pltpu.VMEM((2, CR * K), jnp.int32),
            pltpu.VMEM((2, CR, D), jnp.float32),
            pltpu.VMEM((2, CR, gdim), jnp.float32),
            pltpu.VMEM((2, CR * K, gdim), jnp.float32),
            pltpu.VMEM((2, CR, gdim), jnp.float32),
            pltpu.SemaphoreType.DMA,
            pltpu.SemaphoreType.DMA,
            pltpu.SemaphoreType.DMA,
            pltpu.SemaphoreType.DMA,
            pltpu.SemaphoreType.DMA,
            pltpu.SemaphoreType.DMA,
        ],
    )
    def sc_kernel(gene_hbm, idx_hbm, wsp_hbm, out_hbm,
                  idx_v, w_v, own_v, rows_v, out_v,
                  sa0, sa1, sg0, sg1, so0, so1):
        wid = lax.axis_index("s") * NC + lax.axis_index("c")
        base = wid * rows_per_w
        sa, sg, so = (sa0, sa1), (sg0, sg1), (so0, so1)

        def in_copies(c, b):
            r0 = base + c * CR
            return (
                (idx_hbm.at[pl.ds(r0 * K, CR * K)], idx_v.at[b], sa[b]),
                (wsp_hbm.at[pl.ds(r0, CR)], w_v.at[b], sa[b]),
                (gene_hbm.at[pl.ds(r0, CR)], own_v.at[b], sa[b]),
            )

        def start_in(c, b):
            for src, dst, sem in in_copies(c, b):
                pltpu.async_copy(src, dst, sem)

        def wait_in(c, b):
            for src, dst, sem in in_copies(c, b):
                pltpu.make_async_copy(src, dst, sem).wait()

        def start_g(b):
            pltpu.async_copy(gene_hbm.at[idx_v.at[b]], rows_v.at[b], sg[b])

        def wait_g(b):
            pltpu.make_async_copy(gene_hbm.at[idx_v.at[b]], rows_v.at[b],
                                  sg[b]).wait()

        def out_copy(c, b):
            r0 = base + c * CR
            return (out_v.at[b], out_hbm.at[pl.ds(r0, CR)], so[b])

        def compute(b):
            for r in range(CR):
                wv = [w_v[b, r, pl.ds(k * L, L)] for k in range(K)]

                def dim_step(dc, _):
                    dd = dc * L
                    acc = own_v[b, r, pl.ds(dd, L)]
                    for k in range(K):
                        acc = acc + wv[k] * rows_v[b, r * K + k,
                                                   pl.ds(dd, L)]
                    out_v[b, r, pl.ds(dd, L)] = acc
                    return 0

                lax.fori_loop(0, gdim // L, dim_step, 0)

        start_in(0, 0)
        wait_in(0, 0)
        start_g(0)
        start_in(1, 1)

        def pair(p, _):
            for b in range(2):
                c = 2 * p + b
                nb = 1 - b

                @pl.when(c + 1 < nchunks)
                def _():
                    wait_in(c + 1, nb)
                    start_g(nb)

                wait_g(b)

                @pl.when(c >= 2)
                def _():
                    src, dst, sem = out_copy(c - 2, b)
                    pltpu.make_async_copy(src, dst, sem).wait()

                compute(b)
                src, dst, sem = out_copy(c, b)
                pltpu.async_copy(src, dst, sem)

                @pl.when(c + 2 < nchunks)
                def _():
                    start_in(c + 2, b)

            return 0

        lax.fori_loop(0, nchunks // 2, pair, 0)
        for b in range(2):
            c = nchunks - 2 + b
            src, dst, sem = out_copy(c, b)
            pltpu.make_async_copy(src, dst, sem).wait()

    return sc_kernel(gene, idx_flat, wsp)


NCHUNK = 1


def kernel(raw, img, adata, gene, num):
    n = raw.shape[0]
    x = jnp.concatenate([raw, img, adata], axis=1)
    x = jnp.pad(x, ((0, 0), (0, D - x.shape[1])))
    xt = x.T
    step = n // NCHUNK
    sups, diss = [], []
    token = jnp.int32(0)
    for c in range(NCHUNK):
        dis_c, idx_c, wsp_c = _topk(
            lax.slice_in_dim(x, c * step, (c + 1) * step), xt, c * step)
        sup_c = _combine(gene, idx_c.reshape(-1) + token, wsp_c)
        token = lax.convert_element_type(sup_c[0, 0], jnp.int32) * 0
        sups.append(sup_c)
        diss.append(dis_c)
    return jnp.concatenate(sups, axis=0), jnp.concatenate(diss, axis=0)


# speedup vs baseline: 1.4278x; 1.0277x over previous
import functools

import jax
import jax.numpy as jnp
from jax import lax
from jax.experimental import pallas as pl
from jax.experimental.pallas import tpu as pltpu
from jax.experimental.pallas import tpu_sc as plsc

K = 8
D = 128
BR = 256
NC, NS, L = 2, 16, 16
CR = 8


def _topk_body(row_offset, x_ref, xt_ref, dis_ref, idx_ref, wsp_ref,
               sqc_ref):
    n = xt_ref.shape[1]
    pid = pl.program_id(0)
    x = x_ref[...]
    xt = xt_ref[...]
    sq_r = jnp.sum(x * x, axis=1, keepdims=True)

    @pl.when(pid == 0)
    def _():
        sqc_ref[...] = jnp.sum(xt * xt, axis=0, keepdims=True)

    sq_c = sqc_ref[...]
    g = jnp.dot(x, xt, preferred_element_type=jnp.float32)
    dist = sq_r + sq_c - 2.0 * g
    col = lax.broadcasted_iota(jnp.int32, (BR, n), 1)
    row = (lax.broadcasted_iota(jnp.int32, (BR, n), 0)
           + pid * BR + row_offset)
    dist = jnp.where(col == row, jnp.float32(1e10), dist)
    m0 = jnp.min(dist, axis=1, keepdims=True)
    rel = dist - (m0 - 1.0)
    kb = (lax.bitcast_convert_type(rel, jnp.int32) & jnp.int32(~8191)) | col
    keys = pltpu.bitcast(kb, jnp.float32)
    bigf = jnp.float32(3.0e38)
    dvals, ivals = [], []
    for _ in range(K):
        mk = jnp.min(keys, axis=1, keepdims=True)
        mkb = lax.bitcast_convert_type(mk, jnp.int32)
        dvals.append(m0 - 1.0 + pltpu.bitcast(
            mkb & jnp.int32(~8191), jnp.float32))
        ivals.append(mkb & jnp.int32(8191))
        keys = jnp.where(keys == mk, bigf, keys)
    dis = jnp.concatenate(dvals, axis=1)
    idx = jnp.concatenate(ivals, axis=1)
    dis_ref[...] = dis
    idx_ref[...] = idx
    e = jnp.exp(dis[:, 0:1] - dis)
    w = e / jnp.sum(e, axis=1, keepdims=True)
    kcol = lax.broadcasted_iota(jnp.int32, (BR, D), 1) // L
    wsp = jnp.zeros((BR, D), jnp.float32)
    for k in range(K):
        wsp = jnp.where(kcol == k, w[:, k:k + 1], wsp)
    wsp_ref[...] = wsp


def _topk(xq, xt, row_offset):
    nq = xq.shape[0]
    n = xt.shape[1]
    return pl.pallas_call(
        functools.partial(_topk_body, row_offset),
        grid=(nq // BR,),
        in_specs=[
            pl.BlockSpec((BR, D), lambda i: (i, 0)),
            pl.BlockSpec((D, n), lambda i: (0, 0)),
        ],
        out_specs=[
            pl.BlockSpec((BR, K), lambda i: (i, 0)),
            pl.BlockSpec((BR, K), lambda i: (i, 0)),
            pl.BlockSpec((BR, D), lambda i: (i, 0)),
        ],
        out_shape=[
            jax.ShapeDtypeStruct((nq, K), jnp.float32),
            jax.ShapeDtypeStruct((nq, K), jnp.int32),
            jax.ShapeDtypeStruct((nq, D), jnp.float32),
        ],
        scratch_shapes=[pltpu.VMEM((1, n), jnp.float32)],
    )(xq, xt)


def _combine(gene, idx_flat, wsp):
    gdim = gene.shape[1]
    nq = wsp.shape[0]
    rows_per_w = nq // (NC * NS)
    nchunks = rows_per_w // CR
    mesh = plsc.VectorSubcoreMesh(core_axis_name="c", subcore_axis_name="s",
                                  num_cores=NC, num_subcores=NS)

    @functools.partial(
        pl.kernel,
        out_type=jax.ShapeDtypeStruct((nq, gdim), jnp.float32),
        mesh=mesh,
        scratch_types=[
            pltpu.VMEM((2, CR * K), jnp.int32),
            pltpu.VMEM((2, CR, D), jnp.float32),
            pltpu.VMEM((2, CR, gdim), jnp.float32),
            pltpu.VMEM((2, CR * K, gdim), jnp.float32),
            pltpu.VMEM((2, CR, gdim), jnp.float32),
            pltpu.SemaphoreType.DMA,
            pltpu.SemaphoreType.DMA,
            pltpu.SemaphoreType.DMA,
            pltpu.SemaphoreType.DMA,
            pltpu.SemaphoreType.DMA,
            pltpu.SemaphoreType.DMA,
        ],
    )
    def sc_kernel(gene_hbm, idx_hbm, wsp_hbm, out_hbm,
                  idx_v, w_v, own_v, rows_v, out_v,
                  sa0, sa1, sg0, sg1, so0, so1):
        wid = lax.axis_index("s") * NC + lax.axis_index("c")
        base = wid * rows_per_w
        sa, sg, so = (sa0, sa1), (sg0, sg1), (so0, so1)

        def in_copies(c, b):
            r0 = base + c * CR
            return (
                (idx_hbm.at[pl.ds(r0 * K, CR * K)], idx_v.at[b], sa[b]),
                (wsp_hbm.at[pl.ds(r0, CR)], w_v.at[b], sa[b]),
                (gene_hbm.at[pl.ds(r0, CR)], own_v.at[b], sa[b]),
            )

        def start_in(c, b):
            for src, dst, sem in in_copies(c, b):
                pltpu.async_copy(src, dst, sem)

        def wait_in(c, b):
            for src, dst, sem in in_copies(c, b):
                pltpu.make_async_copy(src, dst, sem).wait()

        def start_g(b):
            pltpu.async_copy(gene_hbm.at[idx_v.at[b]], rows_v.at[b], sg[b])

        def wait_g(b):
            pltpu.make_async_copy(gene_hbm.at[idx_v.at[b]], rows_v.at[b],
                                  sg[b]).wait()

        def out_copy(c, b):
            r0 = base + c * CR
            return (out_v.at[b], out_hbm.at[pl.ds(r0, CR)], so[b])

        def compute(b):
            for r in range(CR):
                wv = [w_v[b, r, pl.ds(k * L, L)] for k in range(K)]

                def dim_step(dc, _):
                    dd = dc * L
                    acc = own_v[b, r, pl.ds(dd, L)]
                    for k in range(K):
                        acc = acc + wv[k] * rows_v[b, r * K + k,
                                                   pl.ds(dd, L)]
                    out_v[b, r, pl.ds(dd, L)] = acc
                    return 0

                lax.fori_loop(0, gdim // L, dim_step, 0)

        start_in(0, 0)
        wait_in(0, 0)
        start_g(0)
        start_in(1, 1)

        def pair(p, _):
            for b in range(2):
                c = 2 * p + b
                nb = 1 - b

                @pl.when(c + 1 < nchunks)
                def _():
                    wait_in(c + 1, nb)
                    start_g(nb)

                wait_g(b)

                @pl.when(c >= 2)
                def _():
                    src, dst, sem = out_copy(c - 2, b)
                    pltpu.make_async_copy(src, dst, sem).wait()

                compute(b)
                src, dst, sem = out_copy(c, b)
                pltpu.async_copy(src, dst, sem)

                @pl.when(c + 2 < nchunks)
                def _():
                    start_in(c + 2, b)

            return 0

        lax.fori_loop(0, nchunks // 2, pair, 0)
        for b in range(2):
            c = nchunks - 2 + b
            src, dst, sem = out_copy(c, b)
            pltpu.make_async_copy(src, dst, sem).wait()

    return sc_kernel(gene, idx_flat, wsp)


NCHUNK = 1


def kernel(raw, img, adata, gene, num):
    n = raw.shape[0]
    x = jnp.concatenate([raw, img, adata], axis=1)
    x = jnp.pad(x, ((0, 0), (0, D - x.shape[1])))
    xt = x.T
    step = n // NCHUNK
    sups, diss = [], []
    token = jnp.int32(0)
    for c in range(NCHUNK):
        dis_c, idx_c, wsp_c = _topk(
            lax.slice_in_dim(x, c * step, (c + 1) * step), xt, c * step)
        sup_c = _combine(gene, idx_c.reshape(-1) + token, wsp_c)
        token = lax.convert_element_type(sup_c[0, 0], jnp.int32) * 0
        sups.append(sup_c)
        diss.append(dis_c)
    return jnp.concatenate(sups, axis=0), jnp.concatenate(diss, axis=0)
